# Initial kernel scaffold; baseline (speedup 1.0000x reference)
import jax
import jax.numpy as jnp
from jax.experimental import pallas as pl


def kernel(logits, targets):
    raise NotImplementedError("write your pallas kernel here")


# speedup vs baseline: 28.5464x; 28.5464x over previous
import functools

import jax
import jax.numpy as jnp
from jax import lax
from jax.experimental import pallas as pl
from jax.experimental.pallas import tpu as pltpu
from jax.experimental.pallas import tpu_sc as plsc

N = 1048576
C = 13
NT = 32
PT = N // NT
CH = 1024
NCH = PT // CH
NB = 2048
SHIFT = 15
BASE = (127 - 8) << 8
HROW = C * NB
HTOT = 3 * HROW

_mesh = plsc.VectorSubcoreMesh(core_axis_name="c", subcore_axis_name="s")


@functools.partial(
    pl.kernel,
    out_type=jax.ShapeDtypeStruct((NT, HTOT), jnp.float32),
    mesh=_mesh,
    compiler_params=pltpu.CompilerParams(needs_layout_passes=False),
    scratch_types=[
        pltpu.VMEM((CH * C,), jnp.float32),
        pltpu.VMEM((CH,), jnp.int32),
        pltpu.VMEM((HTOT,), jnp.float32),
    ],
)
def _hist_kernel(logits_hbm, targets_hbm, out_hbm, lbuf, tbuf, hist):
    wid = lax.axis_index("s") * 2 + lax.axis_index("c")
    pix0 = wid * PT
    lanes = lax.iota(jnp.int32, 16)
    zeros16 = jnp.zeros((16,), jnp.float32)
    ones16 = jnp.ones((16,), jnp.float32)
    shiftv = jnp.full((16,), SHIFT, jnp.int32)

    @pl.loop(0, HTOT // 16)
    def _zero(i):
        hist[pl.ds(i * 16, 16)] = zeros16

    @pl.loop(0, NCH)
    def _chunk(ch):
        base = pix0 + ch * CH
        pltpu.sync_copy(logits_hbm.at[pl.ds(base * C, CH * C)], lbuf)
        pltpu.sync_copy(targets_hbm.at[pl.ds(base, CH)], tbuf)

        @pl.loop(0, CH // 16)
        def _grp(g):
            tgt = tbuf[pl.ds(g * 16, 16)]
            ridx = (g * 16 + lanes) * C
            vs = [plsc.load_gather(lbuf, [ridx + c]) for c in range(C)]
            mx = vs[0]
            for c in range(1, C):
                mx = jnp.maximum(mx, vs[c])
            es = [jnp.exp(v - mx) for v in vs]
            tot = es[0]
            for c in range(1, C):
                tot = tot + es[c]
            inv = 1.0 / tot
            for c in range(C):
                p = es[c] * inv
                isfg = tgt == c
                e = jnp.where(isfg, 1.0 - p, p)
                bits = plsc.bitcast(e, jnp.int32)
                b = lax.shift_right_logical(bits, shiftv) - BASE
                b = jnp.minimum(jnp.maximum(b, 0), NB - 1)
                idx = b + c * NB
                plsc.addupdate_scatter(hist, [idx], ones16)
                plsc.addupdate_scatter(hist, [idx + HROW],
                                       jnp.where(isfg, ones16, zeros16))
                plsc.addupdate_scatter(hist, [idx + 2 * HROW], e)

    pltpu.sync_copy(hist, out_hbm.at[wid])


def _finish_body(hist_ref, out_ref):
    hs = jnp.sum(hist_ref[...], axis=0)
    cnt = hs[0]
    fcn = hs[1]
    esm = hs[2]
    row = lax.broadcasted_iota(jnp.int32, (NB, NB), 0)
    col = lax.broadcasted_iota(jnp.int32, (NB, NB), 1)
    m = (row > col).astype(jnp.float32)
    n_above = jnp.dot(cnt, m, preferred_element_type=jnp.float32,
                      precision=lax.Precision.HIGHEST)
    f_above = jnp.dot(fcn, m, preferred_element_type=jnp.float32,
                      precision=lax.Precision.HIGHEST)
    g = jnp.sum(fcn, axis=1, keepdims=True)

    def jac(n, f):
        den = g + n - f
        return jnp.where(den > 0, 1.0 - (g - f) / jnp.where(den > 0, den, 1.0),
                         0.0)

    dj = jac(n_above + cnt, f_above + fcn) - jac(n_above, f_above)
    ebar = jnp.where(cnt > 0, esm / jnp.where(cnt > 0, cnt, 1.0), 0.0)
    losses = jnp.sum(ebar * dj, axis=1)
    present = g[:, 0] > 0
    countp = jnp.sum(present.astype(jnp.float32))
    total = jnp.sum(jnp.where(present, losses, 0.0))
    res = jnp.where(countp > 0, total / jnp.maximum(countp, 1.0), 0.0)
    out_ref[...] = res.reshape(1, 1)


_finish_kernel = pl.pallas_call(
    _finish_body,
    out_shape=jax.ShapeDtypeStruct((1, 1), jnp.float32),
)


def kernel(logits, targets):
    hist = _hist_kernel(logits.reshape(-1), targets)
    out = _finish_kernel(hist.reshape(NT, 3, C, NB))
    return out.reshape(())


# speedup vs baseline: 29.7375x; 1.0417x over previous
import functools

import jax
import jax.numpy as jnp
from jax import lax
from jax.experimental import pallas as pl
from jax.experimental.pallas import tpu as pltpu
from jax.experimental.pallas import tpu_sc as plsc

N = 1048576
C = 13
NT = 32
PT = N // NT
CH = 1024
NCH = PT // CH
NB = 2048
SHIFT = 15
BASE = (127 - 8) << 8
HROW = C * NB
HTOT = 4 * HROW

_mesh = plsc.VectorSubcoreMesh(core_axis_name="c", subcore_axis_name="s")


@functools.partial(
    pl.kernel,
    out_type=jax.ShapeDtypeStruct((NT, HTOT), jnp.float32),
    mesh=_mesh,
    compiler_params=pltpu.CompilerParams(needs_layout_passes=False),
    scratch_types=[
        pltpu.VMEM((CH * C,), jnp.float32),
        pltpu.VMEM((CH,), jnp.int32),
        pltpu.VMEM((HTOT,), jnp.float32),
    ],
)
def _hist_kernel(logits_hbm, targets_hbm, out_hbm, lbuf, tbuf, hist):
    wid = lax.axis_index("s") * 2 + lax.axis_index("c")
    pix0 = wid * PT
    lanes = lax.iota(jnp.int32, 16)
    zeros16 = jnp.zeros((16,), jnp.float32)
    ones16 = jnp.ones((16,), jnp.float32)
    shiftv = jnp.full((16,), SHIFT, jnp.int32)

    @pl.loop(0, HTOT // 16)
    def _zero(i):
        hist[pl.ds(i * 16, 16)] = zeros16

    @pl.loop(0, NCH)
    def _chunk(ch):
        base = pix0 + ch * CH
        pltpu.sync_copy(logits_hbm.at[pl.ds(base * C, CH * C)], lbuf)
        pltpu.sync_copy(targets_hbm.at[pl.ds(base, CH)], tbuf)

        @pl.loop(0, CH // 16)
        def _grp(g):
            tgt = tbuf[pl.ds(g * 16, 16)]
            ridx = (g * 16 + lanes) * C
            vs = [plsc.load_gather(lbuf, [ridx + c]) for c in range(C)]
            mx = vs[0]
            for c in range(1, C):
                mx = jnp.maximum(mx, vs[c])
            es = [jnp.exp(v - mx) for v in vs]
            tot = es[0]
            for c in range(1, C):
                tot = tot + es[c]
            inv = 1.0 / tot
            for c in range(C):
                p = es[c] * inv
                isfg = tgt == c
                e = jnp.where(isfg, 1.0 - p, p)
                bits = plsc.bitcast(e, jnp.int32)
                b = lax.shift_right_logical(bits, shiftv) - BASE
                b = jnp.minimum(jnp.maximum(b, 0), NB - 1)
                idx = (b + c * NB) + jnp.where(isfg, HROW, 0)
                plsc.addupdate_scatter(hist, [idx], ones16)
                plsc.addupdate_scatter(hist, [idx + 2 * HROW], e)

    pltpu.sync_copy(hist, out_hbm.at[wid])


def _finish_body(hist_ref, out_ref):
    hs = jnp.sum(hist_ref[...], axis=0)
    fcn = hs[1]
    cnt = hs[0] + fcn
    esm = hs[2] + hs[3]
    row = lax.broadcasted_iota(jnp.int32, (NB, NB), 0)
    col = lax.broadcasted_iota(jnp.int32, (NB, NB), 1)
    m = (row > col).astype(jnp.float32)
    n_above = jnp.dot(cnt, m, preferred_element_type=jnp.float32,
                      precision=lax.Precision.HIGHEST)
    f_above = jnp.dot(fcn, m, preferred_element_type=jnp.float32,
                      precision=lax.Precision.HIGHEST)
    g = jnp.sum(fcn, axis=1, keepdims=True)

    def jac(n, f):
        den = g + n - f
        return jnp.where(den > 0, 1.0 - (g - f) / jnp.where(den > 0, den, 1.0),
                         0.0)

    dj = jac(n_above + cnt, f_above + fcn) - jac(n_above, f_above)
    ebar = jnp.where(cnt > 0, esm / jnp.where(cnt > 0, cnt, 1.0), 0.0)
    losses = jnp.sum(ebar * dj, axis=1)
    present = g[:, 0] > 0
    countp = jnp.sum(present.astype(jnp.float32))
    total = jnp.sum(jnp.where(present, losses, 0.0))
    res = jnp.where(countp > 0, total / jnp.maximum(countp, 1.0), 0.0)
    out_ref[...] = res.reshape(1, 1)


_finish_kernel = pl.pallas_call(
    _finish_body,
    out_shape=jax.ShapeDtypeStruct((1, 1), jnp.float32),
)


def kernel(logits, targets):
    hist = _hist_kernel(logits.reshape(-1), targets)
    out = _finish_kernel(hist.reshape(NT, 4, C, NB))
    return out.reshape(())
